# initial kernel scaffold (unmeasured)
import jax
import jax.numpy as jnp
from jax import lax
from jax.experimental import pallas as pl
from jax.experimental.pallas import tpu as pltpu

N_DEV = 4
B, H, D, BS = 8, 8, 64, 16
PAGES_PER_SHARD = 64
K_LOCAL = PAGES_PER_SHARD * BS
NSLOT = 64
PACK = 128
NEG = -1e30


def kernel(Q, K, V, bt, lens):
    Qs = Q.reshape(B, H, D)
    Ks = K.reshape(K_LOCAL, H, D)
    Vs = V.reshape(K_LOCAL, H, D)
    lens2 = lens.reshape(B, 1)

    def body(q_ref, k_ref, v_ref, bt_ref, lens_ref, out_ref,
             acc_ref, comm_ref, send_sems, recv_sems):
        my_pos = lax.axis_index("i")
        left = lax.rem(my_pos + N_DEV - 1, N_DEV)
        right = lax.rem(my_pos + 1, N_DEV)

        barrier_sem = pltpu.get_barrier_semaphore()
        for nbr in (left, right):
            pl.semaphore_signal(barrier_sem, inc=1, device_id=(nbr,),
                                device_id_type=pl.DeviceIdType.MESH)
        pl.semaphore_wait(barrier_sem, 2)

        btv = bt_ref[...]
        lensv = lens_ref[...]
        slot_j = lax.broadcasted_iota(jnp.int32, (B, NSLOT), 1)
        valid = (slot_j < lensv).reshape(B, NSLOT, 1)
        key_page = (lax.broadcasted_iota(jnp.int32, (B, NSLOT, K_LOCAL), 2)
                    // BS) + my_pos * PAGES_PER_SHARD
        eq = (btv.reshape(B, NSLOT, 1) == key_page) & valid
        cnt = jnp.sum(eq.astype(jnp.float32), axis=1)

        qv = q_ref[...]
        kv = k_ref[...]
        vv = v_ref[...]
        scale = jnp.float32(D ** -0.5)
        for h in range(H):
            qh = qv[:, h, :]
            kh = kv[:, h, :]
            vh = vv[:, h, :]
            s = lax.dot_general(qh, kh, (((1,), (1,)), ((), ())),
                                preferred_element_type=jnp.float32) * scale
            s = jnp.where(cnt > 0, s, jnp.float32(NEG))
            m = jnp.max(s, axis=1, keepdims=True)
            e = jnp.exp(s - m) * cnt
            l = jnp.sum(e, axis=1, keepdims=True)
            o = lax.dot_general(e, vh, (((1,), (0,)), ((), ())),
                                preferred_element_type=jnp.float32)
            acc_ref[h, :, 0:D] = o
            acc_ref[h, :, D:D + 1] = m
            acc_ref[h, :, D + 1:D + 2] = l
        comm_ref[0] = acc_ref[...]

        for hop in range(N_DEV - 1):
            rdma = pltpu.make_async_remote_copy(
                src_ref=comm_ref.at[hop],
                dst_ref=comm_ref.at[hop + 1],
                send_sem=send_sems.at[hop],
                recv_sem=recv_sems.at[hop],
                device_id=(right,),
                device_id_type=pl.DeviceIdType.MESH,
            )
            rdma.start()
            rdma.wait()

            o_in = comm_ref[hop + 1, :, :, 0:D]
            m_in = comm_ref[hop + 1, :, :, D:D + 1]
            l_in = comm_ref[hop + 1, :, :, D + 1:D + 2]
            o_r = acc_ref[:, :, 0:D]
            m_r = acc_ref[:, :, D:D + 1]
            l_r = acc_ref[:, :, D + 1:D + 2]
            m_new = jnp.maximum(m_r, m_in)
            a = jnp.exp(m_r - m_new)
            b = jnp.exp(m_in - m_new)
            acc_ref[:, :, 0:D] = o_r * a + o_in * b
            acc_ref[:, :, D:D + 1] = m_new
            acc_ref[:, :, D + 1:D + 2] = l_r * a + l_in * b

        for h in range(H):
            o = acc_ref[h, :, 0:D]
            l = acc_ref[h, :, D + 1:D + 2]
            out_ref[:, 0, h, :] = o / l

    return pl.pallas_call(
        body,
        out_shape=jax.ShapeDtypeStruct((B, 1, H, D), jnp.float32),
        in_specs=[pl.BlockSpec(memory_space=pltpu.VMEM)] * 5,
        out_specs=pl.BlockSpec(memory_space=pltpu.VMEM),
        scratch_shapes=[
            pltpu.VMEM((H, B, PACK), jnp.float32),
            pltpu.VMEM((N_DEV, H, B, PACK), jnp.float32),
            pltpu.SemaphoreType.DMA((N_DEV - 1,)),
            pltpu.SemaphoreType.DMA((N_DEV - 1,)),
        ],
        compiler_params=pltpu.CompilerParams(collective_id=0),
    )(Qs, Ks, Vs, bt, lens2)


# baseline (device time: 48566 ns/iter reference)
import jax
import jax.numpy as jnp
from jax import lax
from jax.experimental import pallas as pl
from jax.experimental.pallas import tpu as pltpu

N_DEV = 4
B, H, D, BS = 8, 8, 64, 16
PAGES_PER_SHARD = 64
K_LOCAL = PAGES_PER_SHARD * BS
NSLOT = 64
PACK = 128
NEG = -1e30


def kernel(Q, K, V, bt, lens):
    Qs = Q.reshape(B, H, D).transpose(1, 0, 2).reshape(H * B, D)
    Ks = K.reshape(K_LOCAL, H, D).transpose(1, 0, 2).reshape(H * K_LOCAL, D)
    Vs = V.reshape(K_LOCAL, H, D).transpose(1, 0, 2).reshape(H * K_LOCAL, D)
    bt3 = bt.reshape(B, 1, NSLOT)
    lens3 = lens.reshape(B, 1, 1)

    def body(q_ref, k_ref, v_ref, bt_ref, lens_ref, out_ref,
             acc_ref, comm_ref, send_sems, recv_sems):
        my_pos = lax.axis_index("i")
        left = lax.rem(my_pos + N_DEV - 1, N_DEV)
        right = lax.rem(my_pos + 1, N_DEV)

        barrier_sem = pltpu.get_barrier_semaphore()
        for nbr in (left, right):
            pl.semaphore_signal(barrier_sem, inc=1, device_id=(nbr,),
                                device_id_type=pl.DeviceIdType.MESH)
        pl.semaphore_wait(barrier_sem, 2)

        slot3 = lax.broadcasted_iota(jnp.int32, (B, K_LOCAL, NSLOT), 2)
        page3 = (lax.broadcasted_iota(jnp.int32, (B, K_LOCAL, NSLOT), 1)
                 // BS) + my_pos * PAGES_PER_SHARD
        eq = (bt_ref[...] == page3) & (slot3 < lens_ref[...])
        cnt = jnp.sum(eq.astype(jnp.float32), axis=2)

        scale = jnp.float32(D ** -0.5)
        for h in range(H):
            qh = q_ref[h * B:(h + 1) * B, :]
            kh = k_ref[h * K_LOCAL:(h + 1) * K_LOCAL, :]
            vh = v_ref[h * K_LOCAL:(h + 1) * K_LOCAL, :]
            s = lax.dot_general(qh, kh, (((1,), (1,)), ((), ())),
                                preferred_element_type=jnp.float32) * scale
            s = jnp.where(cnt > 0, s, jnp.float32(NEG))
            m = jnp.max(s, axis=1, keepdims=True)
            e = jnp.exp(s - m) * cnt
            l = jnp.sum(e, axis=1, keepdims=True)
            o = lax.dot_general(e, vh, (((1,), (0,)), ((), ())),
                                preferred_element_type=jnp.float32)
            acc_ref[h, :, 0:D] = o
            acc_ref[h, :, D:D + 1] = m
            acc_ref[h, :, D + 1:D + 2] = l
        comm_ref[0] = acc_ref[...]

        for hop in range(N_DEV - 1):
            rdma = pltpu.make_async_remote_copy(
                src_ref=comm_ref.at[hop],
                dst_ref=comm_ref.at[hop + 1],
                send_sem=send_sems.at[hop],
                recv_sem=recv_sems.at[hop],
                device_id=(right,),
                device_id_type=pl.DeviceIdType.MESH,
            )
            rdma.start()
            rdma.wait()

            o_in = comm_ref[hop + 1, :, :, 0:D]
            m_in = comm_ref[hop + 1, :, :, D:D + 1]
            l_in = comm_ref[hop + 1, :, :, D + 1:D + 2]
            o_r = acc_ref[:, :, 0:D]
            m_r = acc_ref[:, :, D:D + 1]
            l_r = acc_ref[:, :, D + 1:D + 2]
            m_new = jnp.maximum(m_r, m_in)
            a = jnp.exp(m_r - m_new)
            b = jnp.exp(m_in - m_new)
            acc_ref[:, :, 0:D] = o_r * a + o_in * b
            acc_ref[:, :, D:D + 1] = m_new
            acc_ref[:, :, D + 1:D + 2] = l_r * a + l_in * b

        for h in range(H):
            o = acc_ref[h, :, 0:D]
            l = acc_ref[h, :, D + 1:D + 2]
            out_ref[h, :, :] = o / l

    out_hbd = pl.pallas_call(
        body,
        out_shape=jax.ShapeDtypeStruct((H, B, D), jnp.float32),
        in_specs=[pl.BlockSpec(memory_space=pltpu.VMEM)] * 5,
        out_specs=pl.BlockSpec(memory_space=pltpu.VMEM),
        scratch_shapes=[
            pltpu.VMEM((H, B, PACK), jnp.float32),
            pltpu.VMEM((N_DEV, H, B, PACK), jnp.float32),
            pltpu.SemaphoreType.DMA((N_DEV - 1,)),
            pltpu.SemaphoreType.DMA((N_DEV - 1,)),
        ],
        compiler_params=pltpu.CompilerParams(collective_id=0),
    )(Qs, Ks, Vs, bt3, lens3)
    return out_hbd.transpose(1, 0, 2).reshape(B, 1, H, D)
